# baseline (speedup 1.0000x reference)
import functools

import jax
import jax.numpy as jnp
from jax.experimental import pallas as pl
from jax.experimental.pallas import tpu as pltpu

_F = 26
_E = 16
_B = 4096
_ROW = _F * (_F - 1) * _E
_R = 256


def _pair_offsets():
    pairs = []
    for i in range(_F):
        for j in range(i, _F - 1):
            a = (i * (_F - 1) + j) * _E
            b = ((j + 1) * (_F - 1) + i) * _E
            pairs.append((a, b))
    return pairs


_PAIRS = _pair_offsets()


def _tc_block(x_ref, o_ref):
    x = x_ref[...]
    acc = jnp.zeros((_R, _E), jnp.float32)
    for a, b in _PAIRS:
        acc = acc + x[:, a:a + _E] * x[:, b:b + _E]
    o_ref[...] = jnp.sum(acc, axis=1)


_tc_call = pl.pallas_call(
    _tc_block,
    grid=(_B // _R,),
    in_specs=[pl.BlockSpec((_R, _ROW), lambda i: (i, 0))],
    out_specs=pl.BlockSpec((_R,), lambda i: (i,)),
    out_shape=jax.ShapeDtypeStruct((_B,), jnp.float32),
    compiler_params=pltpu.CompilerParams(
        dimension_semantics=("arbitrary",)),
)


def kernel(inputs):
    return _tc_call(inputs)


# speedup vs baseline: 1.2285x; 1.2285x over previous
import functools

import jax
import jax.numpy as jnp
from jax import lax
from jax.experimental import pallas as pl
from jax.experimental.pallas import tpu as pltpu
from jax.experimental.pallas import tpu_sc as plsc

_F = 26
_E = 16
_B = 4096
_ROW = _F * (_F - 1) * _E

_NC = 2
_NS = 16
_NW = _NC * _NS

_SC_ROWS = 1536
_RPW = _SC_ROWS // _NW
_NACC = 8
_CH = 4

_R = 256
_TC_ROWS = _B - _SC_ROWS


def _pair_offsets():
    pairs = []
    for i in range(_F):
        for j in range(i, _F - 1):
            a = (i * (_F - 1) + j) * _E
            b = ((j + 1) * (_F - 1) + i) * _E
            pairs.append((a, b))
    return pairs


_PAIRS = _pair_offsets()


def _row_reduce(buf, rl):
    accs = [jnp.zeros((_E,), jnp.float32) for _ in range(_NACC)]
    for k, (a, b) in enumerate(_PAIRS):
        accs[k % _NACC] += buf[rl, pl.ds(a, _E)] * buf[rl, pl.ds(b, _E)]
    tot = accs[0]
    for v in accs[1:]:
        tot = tot + v
    return tot


def _tec_body(x_hbm, out_hbm, buf0, buf1, tots_v, out_v, sem0, sem1):
    wid = lax.axis_index("s") * _NC + lax.axis_index("c")
    base = wid * _RPW
    bufs = (buf0, buf1)
    sems = (sem0, sem1)
    nchunks = _RPW // _CH

    pltpu.async_copy(x_hbm.at[pl.ds(base, _CH)], buf0, sem0)

    def step(g, _):
        for p in range(2):
            ch = g * 2 + p

            @pl.when(ch + 1 < nchunks)
            def _():
                pltpu.async_copy(
                    x_hbm.at[pl.ds(base + (ch + 1) * _CH, _CH)],
                    bufs[1 - p], sems[1 - p])

            pltpu.make_async_copy(
                x_hbm.at[pl.ds(base + ch * _CH, _CH)],
                bufs[p], sems[p]).wait()

            def row(rl, _):
                tot = _row_reduce(bufs[p], rl)
                tots_v[pl.ds((ch * _CH + rl) * _E, _E)] = tot
                return 0

            lax.fori_loop(0, _CH, row, 0)
        return 0

    lax.fori_loop(0, nchunks // 2, step, 0)

    rows16 = jnp.arange(_E, dtype=jnp.int32)
    for g in range(_RPW // _E):
        idx0 = (rows16 + g * _E) * _E
        acc = plsc.load_gather(tots_v, [idx0])
        for e in range(1, _E):
            acc += plsc.load_gather(tots_v, [idx0 + e])
        out_v[pl.ds(g * _E, _E)] = acc
    pltpu.sync_copy(out_v, out_hbm.at[pl.ds(base, _RPW)])


@functools.partial(
    pl.kernel,
    out_type=jax.ShapeDtypeStruct((_SC_ROWS,), jnp.float32),
    mesh=plsc.VectorSubcoreMesh(
        core_axis_name="c", subcore_axis_name="s",
        num_cores=_NC, num_subcores=_NS),
    compiler_params=pltpu.CompilerParams(needs_layout_passes=False),
    scratch_types=[
        pltpu.VMEM((_CH, _ROW), jnp.float32),
        pltpu.VMEM((_CH, _ROW), jnp.float32),
        pltpu.VMEM((_RPW * _E,), jnp.float32),
        pltpu.VMEM((_RPW,), jnp.float32),
        pltpu.SemaphoreType.DMA,
        pltpu.SemaphoreType.DMA,
    ],
)
def _fm_sc_kernel(x_hbm, out_hbm, buf0, buf1, tots_v, out_v, sem0, sem1):
    _tec_body(x_hbm, out_hbm, buf0, buf1, tots_v, out_v, sem0, sem1)


def _tc_block(x_ref, o_ref):
    x = x_ref[...]
    acc = jnp.zeros((_R, _E), jnp.float32)
    for a, b in _PAIRS:
        acc = acc + x[:, a:a + _E] * x[:, b:b + _E]
    o_ref[...] = jnp.sum(acc, axis=1)


_tc_call = pl.pallas_call(
    _tc_block,
    grid=(_TC_ROWS // _R,),
    in_specs=[pl.BlockSpec((_R, _ROW), lambda i: (_SC_ROWS // _R + i, 0))],
    out_specs=pl.BlockSpec((_R,), lambda i: (i,)),
    out_shape=jax.ShapeDtypeStruct((_TC_ROWS,), jnp.float32),
    compiler_params=pltpu.CompilerParams(
        dimension_semantics=("arbitrary",)),
)


def kernel(inputs):
    out_sc = _fm_sc_kernel(inputs)
    out_tc = _tc_call(inputs)
    return jnp.concatenate([out_sc, out_tc])
